# baseline (speedup 1.0000x reference)
import jax
import jax.numpy as jnp
from jax.experimental import pallas as pl
from jax.experimental.pallas import tpu as pltpu

B, T, V = 64, 64, 128
D = 16
QD, NE, PLM = 5, 8, 768
DS, NC = 9, 2
RARITY_ALPHA = 0.5
K = 8
G = B // K
C = 3 * D
QC = QD * C


def _kedgn_kernel(
    data_ref, mask_ref, ai_ref, ait_ref, len_ref, bt_ref, plm_ref, pstat_ref,
    wvt_ref, bvt_ref, wpert_ref, bpert_ref, wlin_ref, blin_ref,
    tembt_ref, adj_ref, rwt_ref,
    wf_ref, bf_ref, wg_ref, bg_ref,
    pzrht_ref, phht_ref, bzrt_ref, bht_ref,
    wemb_ref, bemb_ref, wc1_ref, bc1_ref, wc2_ref, bc2_ref,
    out_ref, agg_ref, fus_ref,
):
    f32 = jnp.float32
    bf16 = jnp.bfloat16

    def dot(a, b):
        return jnp.dot(a, b, preferred_element_type=f32)

    rwt = rwt_ref[...]
    mrw = jnp.max(jnp.abs(rwt))
    type_embt = tembt_ref[...]
    wvt = wvt_ref[...]
    bvt = bvt_ref[...]
    pzrht = pzrht_ref[...].astype(bf16)
    phht = phht_ref[...].astype(bf16)
    bzrt = bzrt_ref[...]
    bht = bht_ref[...]

    qT = []
    qTb = []
    attn_s = []
    lens = []
    rc_parts = []
    te_parts = []
    for k in range(K):
        plm_k = plm_ref[k]
        q_k = jnp.tanh(dot(plm_k, wf_ref[...]) + bf_ref[...])
        ne_k = dot(plm_k, wg_ref[...]) + bg_ref[...]
        qT.append(q_k.T)
        qTb.append(q_k.T.astype(bf16))
        attn_k = jax.nn.relu(dot(ne_k, ne_k.T)) * adj_ref[...]
        mxc = jnp.max(attn_k, axis=0, keepdims=True) + mrw
        attn_s.append(attn_k - mxc)
        lens.append(len_ref[k, 0, 0])
        rc_parts.append(RARITY_ALPHA * jnp.tanh(ait_ref[k]))
        tvec = bt_ref[k]
        te_parts.append(jnp.concatenate(
            [tvec * wlin_ref[0, 0] + blin_ref[0, 0],
             jnp.sin(wpert_ref[...] * tvec + bpert_ref[...])], axis=0))

    bank = jnp.concatenate(rc_parts + te_parts, axis=0)


    maxlen = lens[0]
    for k in range(1, K):
        maxlen = jnp.maximum(maxlen, lens[k])

    def sub_step(t, h_all):
        onehot = (jax.lax.broadcasted_iota(jnp.int32, (T, 1), 0) == t
                  ).astype(f32)
        cols = dot(bank, onehot)
        rc_all = cols[:K * V]
        te_all = cols[K * V:]

        xs, msgs, hs, mes, u2s = [], [], [], [], []
        for k in range(K):
            valid = jnp.where(t < lens[k], 1.0, 0.0).astype(f32)
            m_row = mask_ref[k, pl.ds(t, 1), :]
            me_row = m_row * valid
            d_row = data_ref[k, pl.ds(t, 1), :]
            ai_row = ai_ref[k, pl.ds(t, 1), :]
            rar_row = RARITY_ALPHA * jnp.tanh(ai_row)
            rc_k = rc_all[k * V:(k + 1) * V]
            e = jnp.exp(attn_s[k] - rwt * (rc_k + rar_row))
            h_k = h_all[k * D:(k + 1) * D]
            sm = dot(jnp.concatenate([me_row, h_k * me_row], axis=0), e)
            sp = sm[:1]
            msg = sm[1:] * (me_row / (sp + 1e-30))
            x_k = (d_row * wvt + bvt + te_all[k * D:(k + 1) * D]
                   + type_embt) * m_row
            u = jnp.concatenate([x_k, msg, h_k], axis=0).astype(bf16)
            u2 = jnp.concatenate([qTb[k][j:j + 1] * u for j in range(QD)],
                                 axis=0)
            xs.append(x_k); msgs.append(msg); hs.append(h_k)
            mes.append(me_row); u2s.append(u2)

        u2_all = jnp.concatenate(u2s, axis=1)
        zr_all = jax.nn.sigmoid(dot(pzrht, u2_all) + bzrt)

        ug2s = []
        for k in range(K):
            r_k = zr_all[D:, k * V:(k + 1) * V]
            ug = jnp.concatenate(
                [xs[k], msgs[k], r_k * hs[k]], axis=0).astype(bf16)
            ug2s.append(jnp.concatenate(
                [qTb[k][j:j + 1] * ug for j in range(QD)], axis=0))
        ug2_all = jnp.concatenate(ug2s, axis=1)
        hh_all = jnp.tanh(dot(phht, ug2_all) + bht)

        h_next = []
        for k in range(K):
            z_k = zr_all[:D, k * V:(k + 1) * V]
            hh_k = hh_all[:, k * V:(k + 1) * V]
            h_new = (1.0 - z_k) * hs[k] + z_k * hh_k
            h_next.append(jnp.where(mes[k] > 0, h_new, hs[k]))
        return jnp.concatenate(h_next, axis=0)

    UNROLL = 4

    def stepu(i, h_all):
        for j in range(UNROLL):
            h_all = sub_step(UNROLL * i + j, h_all)
        return h_all

    h0 = jnp.zeros((K * D, V), f32)
    h_last = jax.lax.fori_loop(
        0, (maxlen + UNROLL - 1) // UNROLL, stepu, h0)

    agg = jnp.concatenate(
        [jnp.sum(h_last[k * D:(k + 1) * D], axis=0, keepdims=True)
         for k in range(K)], axis=0)
    pst = jnp.concatenate([pstat_ref[k] for k in range(K)], axis=0)
    se = dot(pst, wemb_ref[...]) + bemb_ref[...]
    fused = jnp.concatenate([agg, se], axis=1)
    hid = jax.nn.relu(dot(fused, wc1_ref[...]) + bc1_ref[...])
    out = dot(hid, wc2_ref[...]) + bc2_ref[...]

    out_ref[0] = out
    agg_ref[0] = agg
    fus_ref[0] = fused


def kernel(P, P_static, P_avg_interval, P_length, P_time, P_var_plm_rep_tensor,
           W_val, b_val, W_per, b_per, W_lin, b_lin, type_emb, adj, rarity_W,
           W_f, b_f, W_g, b_g, P_z, b_z, P_r, b_r, P_h, b_h,
           W_emb, b_emb, W_c1, b_c1, W_c2, b_c2):
    f32 = jnp.float32
    len_i32 = P_length[:, 0].astype(jnp.int32)
    perm = jnp.argsort(len_i32)
    inv_perm = jnp.argsort(perm)
    take = lambda x: jnp.take(x, perm, axis=0)
    Pp = take(P)
    data = Pp[:, :, :V]
    mask = Pp[:, :, V:]
    ai = take(P_avg_interval)
    ait = jnp.swapaxes(ai, 1, 2)
    base_t = take(P_time[:, :, 0]).reshape(B, 1, T)
    lengths = take(len_i32).reshape(B, 1, 1)
    plm = take(P_var_plm_rep_tensor)
    pstat = take(P_static).reshape(B, 1, DS)
    pz2 = P_z.reshape(QC, D)
    pr2 = P_r.reshape(QC, D)
    pzrht = jnp.concatenate([pz2, pr2], axis=-1).T
    phht = P_h.reshape(QC, D).T
    bzrt = jnp.concatenate([b_z, b_r]).reshape(2 * D, 1)

    def col(x):
        return x.reshape(-1, 1).astype(f32)

    def row(x):
        return x.reshape(1, -1).astype(f32)

    per_g3 = lambda s1, s2: pl.BlockSpec((K, s1, s2), lambda g: (g, 0, 0))
    full2 = lambda s1, s2: pl.BlockSpec((s1, s2), lambda g: (0, 0))

    out_shapes = (
        jax.ShapeDtypeStruct((G, K, NC), f32),
        jax.ShapeDtypeStruct((G, K, V), f32),
        jax.ShapeDtypeStruct((G, K, V + D), f32),
    )
    out_specs = (per_g3(K, NC), per_g3(K, V), per_g3(K, V + D))
    out_specs = (pl.BlockSpec((1, K, NC), lambda g: (g, 0, 0)),
                 pl.BlockSpec((1, K, V), lambda g: (g, 0, 0)),
                 pl.BlockSpec((1, K, V + D), lambda g: (g, 0, 0)))

    outs = pl.pallas_call(
        _kedgn_kernel,
        grid=(G,),
        in_specs=[
            per_g3(T, V),
            per_g3(T, V),
            per_g3(T, V),
            per_g3(V, T),
            pl.BlockSpec((K, 1, 1), lambda g: (g, 0, 0),
                         memory_space=pltpu.SMEM),
            per_g3(1, T),
            per_g3(V, PLM),
            per_g3(1, DS),
            full2(D, 1), full2(D, 1),
            full2(D - 1, 1), full2(D - 1, 1),
            full2(1, 1), full2(1, 1),
            full2(D, V),
            full2(V, V),
            full2(V, V),
            full2(PLM, QD), full2(1, QD),
            full2(PLM, NE), full2(1, NE),
            full2(2 * D, QC),
            full2(D, QC),
            full2(2 * D, 1), full2(D, 1),
            full2(DS, D), full2(1, D),
            full2(V + D, D), full2(1, D),
            full2(D, NC), full2(1, NC),
        ],
        out_specs=out_specs,
        out_shape=out_shapes,
    )(
        data, mask, ai, ait, lengths, base_t, plm, pstat,
        col(W_val), col(b_val), col(W_per), col(b_per),
        W_lin.astype(f32), b_lin.reshape(1, 1),
        type_emb.T, adj, rarity_W.T,
        W_f, row(b_f), W_g, row(b_g),
        pzrht, phht, bzrt, col(b_h),
        W_emb, row(b_emb), W_c1, row(b_c1), W_c2, row(b_c2),
    )
    output, aggregated_hidden, fused_features = outs
    unperm = lambda x: jnp.take(x, inv_perm, axis=0)
    return (unperm(output.reshape(B, NC)),
            unperm(aggregated_hidden.reshape(B, V)),
            unperm(fused_features.reshape(B, V + D)))


# speedup vs baseline: 1.3014x; 1.3014x over previous
import jax
import jax.numpy as jnp
from jax.experimental import pallas as pl
from jax.experimental.pallas import tpu as pltpu

B, T, V = 64, 64, 128
D = 16
QD, NE, PLM = 5, 8, 768
DS, NC = 9, 2
RARITY_ALPHA = 0.5
K = 32
G = B // K
C = 3 * D
QC = QD * C


def _kedgn_kernel(
    data_ref, mask_ref, ai_ref, ait_ref, len_ref, bt_ref, plm_ref, pstat_ref,
    wvt_ref, bvt_ref, wpert_ref, bpert_ref, wlin_ref, blin_ref,
    tembt_ref, adj_ref, rwt_ref,
    wf_ref, bf_ref, wg_ref, bg_ref,
    pzrht_ref, phht_ref, bzrt_ref, bht_ref,
    wemb_ref, bemb_ref, wc1_ref, bc1_ref, wc2_ref, bc2_ref,
    out_ref, agg_ref, fus_ref,
):
    f32 = jnp.float32
    bf16 = jnp.bfloat16

    def dot(a, b):
        return jnp.dot(a, b, preferred_element_type=f32)

    rwt = rwt_ref[...]
    mrw = jnp.max(jnp.abs(rwt))
    type_embt = tembt_ref[...]
    wvt = wvt_ref[...]
    bvt = bvt_ref[...]
    pzrht = pzrht_ref[...].astype(bf16)
    phht = phht_ref[...].astype(bf16)
    bzrt = bzrt_ref[...]
    bht = bht_ref[...]

    qT = []
    qTb = []
    attn_s = []
    lens = []
    rc_parts = []
    te_parts = []
    for k in range(K):
        plm_k = plm_ref[k]
        q_k = jnp.tanh(dot(plm_k, wf_ref[...]) + bf_ref[...])
        ne_k = dot(plm_k, wg_ref[...]) + bg_ref[...]
        qT.append(q_k.T)
        qTb.append(q_k.T.astype(bf16))
        attn_k = jax.nn.relu(dot(ne_k, ne_k.T)) * adj_ref[...]
        mxc = jnp.max(attn_k, axis=0, keepdims=True) + mrw
        attn_s.append(attn_k - mxc)
        lens.append(len_ref[k, 0, 0])
        rc_parts.append(RARITY_ALPHA * jnp.tanh(ait_ref[k]))
        tvec = bt_ref[k]
        te_parts.append(jnp.concatenate(
            [tvec * wlin_ref[0, 0] + blin_ref[0, 0],
             jnp.sin(wpert_ref[...] * tvec + bpert_ref[...])], axis=0))

    bank = jnp.concatenate(rc_parts + te_parts, axis=0)


    maxlen = lens[0]
    for k in range(1, K):
        maxlen = jnp.maximum(maxlen, lens[k])

    def sub_step(t, h_all):
        onehot = (jax.lax.broadcasted_iota(jnp.int32, (T, 1), 0) == t
                  ).astype(f32)
        cols = dot(bank, onehot)
        rc_all = cols[:K * V]
        te_all = cols[K * V:]

        xs, msgs, hs, mes, u2s = [], [], [], [], []
        for k in range(K):
            valid = jnp.where(t < lens[k], 1.0, 0.0).astype(f32)
            m_row = mask_ref[k, pl.ds(t, 1), :]
            me_row = m_row * valid
            d_row = data_ref[k, pl.ds(t, 1), :]
            ai_row = ai_ref[k, pl.ds(t, 1), :]
            rar_row = RARITY_ALPHA * jnp.tanh(ai_row)
            rc_k = rc_all[k * V:(k + 1) * V]
            e = jnp.exp(attn_s[k] - rwt * (rc_k + rar_row))
            h_k = h_all[k * D:(k + 1) * D]
            sm = dot(jnp.concatenate([me_row, h_k * me_row], axis=0), e)
            sp = sm[:1]
            msg = sm[1:] * (me_row / (sp + 1e-30))
            x_k = (d_row * wvt + bvt + te_all[k * D:(k + 1) * D]
                   + type_embt) * m_row
            u = jnp.concatenate([x_k, msg, h_k], axis=0).astype(bf16)
            u2 = jnp.concatenate([qTb[k][j:j + 1] * u for j in range(QD)],
                                 axis=0)
            xs.append(x_k); msgs.append(msg); hs.append(h_k)
            mes.append(me_row); u2s.append(u2)

        u2_all = jnp.concatenate(u2s, axis=1)
        zr_all = jax.nn.sigmoid(dot(pzrht, u2_all) + bzrt)

        ug2s = []
        for k in range(K):
            r_k = zr_all[D:, k * V:(k + 1) * V]
            ug = jnp.concatenate(
                [xs[k], msgs[k], r_k * hs[k]], axis=0).astype(bf16)
            ug2s.append(jnp.concatenate(
                [qTb[k][j:j + 1] * ug for j in range(QD)], axis=0))
        ug2_all = jnp.concatenate(ug2s, axis=1)
        hh_all = jnp.tanh(dot(phht, ug2_all) + bht)

        h_next = []
        for k in range(K):
            z_k = zr_all[:D, k * V:(k + 1) * V]
            hh_k = hh_all[:, k * V:(k + 1) * V]
            h_new = (1.0 - z_k) * hs[k] + z_k * hh_k
            h_next.append(jnp.where(mes[k] > 0, h_new, hs[k]))
        return jnp.concatenate(h_next, axis=0)

    UNROLL = 4

    def stepu(i, h_all):
        for j in range(UNROLL):
            h_all = sub_step(UNROLL * i + j, h_all)
        return h_all

    h0 = jnp.zeros((K * D, V), f32)
    h_last = jax.lax.fori_loop(
        0, (maxlen + UNROLL - 1) // UNROLL, stepu, h0)

    agg = jnp.concatenate(
        [jnp.sum(h_last[k * D:(k + 1) * D], axis=0, keepdims=True)
         for k in range(K)], axis=0)
    pst = jnp.concatenate([pstat_ref[k] for k in range(K)], axis=0)
    se = dot(pst, wemb_ref[...]) + bemb_ref[...]
    fused = jnp.concatenate([agg, se], axis=1)
    hid = jax.nn.relu(dot(fused, wc1_ref[...]) + bc1_ref[...])
    out = dot(hid, wc2_ref[...]) + bc2_ref[...]

    out_ref[0] = out
    agg_ref[0] = agg
    fus_ref[0] = fused


def kernel(P, P_static, P_avg_interval, P_length, P_time, P_var_plm_rep_tensor,
           W_val, b_val, W_per, b_per, W_lin, b_lin, type_emb, adj, rarity_W,
           W_f, b_f, W_g, b_g, P_z, b_z, P_r, b_r, P_h, b_h,
           W_emb, b_emb, W_c1, b_c1, W_c2, b_c2):
    f32 = jnp.float32
    len_i32 = P_length[:, 0].astype(jnp.int32)
    perm = jnp.argsort(len_i32)
    inv_perm = jnp.argsort(perm)
    take = lambda x: jnp.take(x, perm, axis=0)
    Pp = take(P)
    data = Pp[:, :, :V]
    mask = Pp[:, :, V:]
    ai = take(P_avg_interval)
    ait = jnp.swapaxes(ai, 1, 2)
    base_t = take(P_time[:, :, 0]).reshape(B, 1, T)
    lengths = take(len_i32).reshape(B, 1, 1)
    plm = take(P_var_plm_rep_tensor)
    pstat = take(P_static).reshape(B, 1, DS)
    pz2 = P_z.reshape(QC, D)
    pr2 = P_r.reshape(QC, D)
    pzrht = jnp.concatenate([pz2, pr2], axis=-1).T
    phht = P_h.reshape(QC, D).T
    bzrt = jnp.concatenate([b_z, b_r]).reshape(2 * D, 1)

    def col(x):
        return x.reshape(-1, 1).astype(f32)

    def row(x):
        return x.reshape(1, -1).astype(f32)

    per_g3 = lambda s1, s2: pl.BlockSpec((K, s1, s2), lambda g: (g, 0, 0))
    full2 = lambda s1, s2: pl.BlockSpec((s1, s2), lambda g: (0, 0))

    out_shapes = (
        jax.ShapeDtypeStruct((G, K, NC), f32),
        jax.ShapeDtypeStruct((G, K, V), f32),
        jax.ShapeDtypeStruct((G, K, V + D), f32),
    )
    out_specs = (per_g3(K, NC), per_g3(K, V), per_g3(K, V + D))
    out_specs = (pl.BlockSpec((1, K, NC), lambda g: (g, 0, 0)),
                 pl.BlockSpec((1, K, V), lambda g: (g, 0, 0)),
                 pl.BlockSpec((1, K, V + D), lambda g: (g, 0, 0)))

    outs = pl.pallas_call(
        _kedgn_kernel,
        grid=(G,),
        in_specs=[
            per_g3(T, V),
            per_g3(T, V),
            per_g3(T, V),
            per_g3(V, T),
            pl.BlockSpec((K, 1, 1), lambda g: (g, 0, 0),
                         memory_space=pltpu.SMEM),
            per_g3(1, T),
            per_g3(V, PLM),
            per_g3(1, DS),
            full2(D, 1), full2(D, 1),
            full2(D - 1, 1), full2(D - 1, 1),
            full2(1, 1), full2(1, 1),
            full2(D, V),
            full2(V, V),
            full2(V, V),
            full2(PLM, QD), full2(1, QD),
            full2(PLM, NE), full2(1, NE),
            full2(2 * D, QC),
            full2(D, QC),
            full2(2 * D, 1), full2(D, 1),
            full2(DS, D), full2(1, D),
            full2(V + D, D), full2(1, D),
            full2(D, NC), full2(1, NC),
        ],
        out_specs=out_specs,
        out_shape=out_shapes,
    )(
        data, mask, ai, ait, lengths, base_t, plm, pstat,
        col(W_val), col(b_val), col(W_per), col(b_per),
        W_lin.astype(f32), b_lin.reshape(1, 1),
        type_emb.T, adj, rarity_W.T,
        W_f, row(b_f), W_g, row(b_g),
        pzrht, phht, bzrt, col(b_h),
        W_emb, row(b_emb), W_c1, row(b_c1), W_c2, row(b_c2),
    )
    output, aggregated_hidden, fused_features = outs
    unperm = lambda x: jnp.take(x, inv_perm, axis=0)
    return (unperm(output.reshape(B, NC)),
            unperm(aggregated_hidden.reshape(B, V)),
            unperm(fused_features.reshape(B, V + D)))
